# baseline (speedup 1.0000x reference)
import functools

import jax
import jax.numpy as jnp
from jax import lax
from jax.experimental import pallas as pl
from jax.experimental.pallas import tpu as pltpu
from jax.experimental.pallas import tpu_sc as plsc

N_NODES = 10000
N_ACC = 10112
E_EDGES = 160000
EB = 512
NB = 320
E_PAD = NB * EB
NFEAT = 256
NHID = 512
NCLASS = 47
W = 64
CW = 64
ROWS_PER_TILE = N_ACC // 16
BM = 632


def _make_propagate(n_chunks, split_edges, n_out, wp=W, tc_tiling=True):
    mesh = plsc.VectorSubcoreMesh(core_axis_name="c", subcore_axis_name="s")
    count = NB // 32 if split_edges else NB // 16
    params = pltpu.CompilerParams(use_tc_tiling_on_sc=tc_tiling)

    @functools.partial(
        pl.kernel,
        mesh=mesh,
        compiler_params=params,
        out_type=jax.ShapeDtypeStruct((n_out, N_ACC, wp), jnp.float32),
        scratch_types=[
            pltpu.VMEM((2, EB), jnp.int32),
            pltpu.VMEM((2, EB), jnp.int32),
            pltpu.VMEM((EB, wp), jnp.float32),
            pltpu.VMEM((EB, wp), jnp.float32),
            pltpu.VMEM_SHARED((N_ACC, wp), jnp.float32),
            pltpu.SemaphoreType.DMA,
            pltpu.SemaphoreType.DMA,
            pltpu.SemaphoreType.DMA,
        ],
    )
    def prop(table, edges_b, zeros_hbm, out,
             idxA, idxB, rowsA, rowsB, accum,
             semA, semB, semI):
        cid = lax.axis_index("c")
        sid = lax.axis_index("s")
        if split_edges:
            base = cid * (NB // 2) + sid * count
        else:
            base = sid * count
        row0 = sid * ROWS_PER_TILE
        hops = [(0, 128), (128, 128), (256, 128), (384, 128), (512, 120)]

        def wait_rows(buf, sem):
            pltpu.make_async_copy(zeros_hbm, buf, sem).wait()

        def wait_idx(buf, sem):
            pltpu.make_async_copy(edges_b.at[0], buf, sem).wait()

        for c in range(n_chunks if not split_edges else 2):
            owner = (c % 2) if not split_edges else c

            @pl.when(cid == owner)
            def _round(c=c):
                pltpu.sync_copy(zeros_hbm, rowsA)
                for off, sz in hops:
                    pltpu.sync_copy(rowsA.at[pl.ds(0, sz)],
                                    accum.at[pl.ds(row0 + off, sz)])
                plsc.subcore_barrier()
                tbl = table.at[0 if split_edges else c]
                pltpu.sync_copy(edges_b.at[base], idxA)
                pltpu.sync_copy(edges_b.at[base + 1], idxB)
                pltpu.async_copy(tbl.at[idxA.at[0]], rowsA, semA)

                def pair(i, carry):
                    j2 = jnp.minimum(2 * i + 2, count - 1)
                    j3 = jnp.minimum(2 * i + 3, count - 1)
                    pltpu.async_copy(tbl.at[idxB.at[0]], rowsB, semB)
                    wait_rows(rowsA, semA)
                    pltpu.sync_copy(rowsA, accum.at[idxA.at[1]],
                                    add=True)
                    pltpu.async_copy(edges_b.at[base + j2], idxA, semI)
                    wait_rows(rowsB, semB)
                    wait_idx(idxA, semI)
                    pltpu.async_copy(tbl.at[idxA.at[0]], rowsA, semA)
                    pltpu.sync_copy(rowsB, accum.at[idxB.at[1]],
                                    add=True)
                    pltpu.sync_copy(edges_b.at[base + j3], idxB)
                    return carry

                lax.fori_loop(0, count // 2, pair, 0)
                wait_rows(rowsA, semA)
                plsc.subcore_barrier()
                for off, sz in hops:
                    rows = pl.ds(row0 + off, sz)
                    pltpu.sync_copy(accum.at[rows],
                                    rowsB.at[pl.ds(0, sz)])
                    pltpu.sync_copy(rowsB.at[pl.ds(0, sz)],
                                    out.at[c].at[rows])

    return prop


_prop_l1 = _make_propagate(n_chunks=4, split_edges=False, n_out=4,
                           tc_tiling=False)
_prop_l2 = _make_propagate(n_chunks=8, split_edges=False, n_out=8,
                           tc_tiling=False)
_prop_l3 = _make_propagate(n_chunks=1, split_edges=True, n_out=2,
                           wp=CW, tc_tiling=False)


def _t1_body(p_ref, w_ref, b_ref, o_ref):
    acc = jnp.dot(p_ref[0], w_ref[:W, :], preferred_element_type=jnp.float32)
    for k in range(1, 4):
        acc = acc + jnp.dot(p_ref[k], w_ref[k * W:(k + 1) * W, :],
                            preferred_element_type=jnp.float32)
    res = jnp.maximum(acc + b_ref[...], 0.0)
    for c in range(8):
        o_ref[c] = res[:, c * W:(c + 1) * W]


def _t2_body(p_ref, w2_ref, b2_ref, w3_ref, o_ref):
    acc = jnp.dot(p_ref[0], w2_ref[:W, :], preferred_element_type=jnp.float32)
    for k in range(1, 8):
        acc = acc + jnp.dot(p_ref[k], w2_ref[k * W:(k + 1) * W, :],
                            preferred_element_type=jnp.float32)
    h = jnp.maximum(acc + b2_ref[...], 0.0)
    o_ref[...] = jnp.dot(h, w3_ref[...], preferred_element_type=jnp.float32)


def _t3_body(p_ref, b_ref, o_ref):
    s = p_ref[0] + p_ref[1] + b_ref[...]
    col = lax.broadcasted_iota(jnp.int32, s.shape, 1)
    valid = col < NCLASS
    m = jnp.max(jnp.where(valid, s, -1e30), axis=1, keepdims=True)
    e = jnp.where(valid, jnp.exp(s - m), 0.0)
    lse = jnp.log(jnp.sum(e, axis=1, keepdims=True)) + m
    o_ref[...] = (s - lse)[:, :NCLASS]


def _tc_matmul1(p1, W1, b1):
    grid = (N_ACC // BM,)
    return pl.pallas_call(
        _t1_body,
        grid=grid,
        in_specs=[
            pl.BlockSpec((4, BM, W), lambda m: (0, m, 0)),
            pl.BlockSpec((NFEAT, NHID), lambda m: (0, 0)),
            pl.BlockSpec((1, NHID), lambda m: (0, 0)),
        ],
        out_specs=pl.BlockSpec((8, BM, W), lambda m: (0, m, 0)),
        out_shape=jax.ShapeDtypeStruct((8, N_ACC, W), jnp.float32),
    )(p1, W1, b1.reshape(1, NHID))


def _tc_matmul2(p2, W2, b2, W3p):
    grid = (N_ACC // BM,)
    return pl.pallas_call(
        _t2_body,
        grid=grid,
        in_specs=[
            pl.BlockSpec((8, BM, W), lambda m: (0, m, 0)),
            pl.BlockSpec((NHID, NHID), lambda m: (0, 0)),
            pl.BlockSpec((1, NHID), lambda m: (0, 0)),
            pl.BlockSpec((NHID, CW), lambda m: (0, 0)),
        ],
        out_specs=pl.BlockSpec((BM, CW), lambda m: (m, 0)),
        out_shape=jax.ShapeDtypeStruct((N_ACC, CW), jnp.float32),
    )(p2, W2, b2.reshape(1, NHID), W3p)


def _tc_logsoftmax(p3, b3p):
    grid = (N_ACC // BM,)
    return pl.pallas_call(
        _t3_body,
        grid=grid,
        in_specs=[
            pl.BlockSpec((2, BM, CW), lambda m: (0, m, 0)),
            pl.BlockSpec((1, CW), lambda m: (0, 0)),
        ],
        out_specs=pl.BlockSpec((BM, NCLASS), lambda m: (m, 0)),
        out_shape=jax.ShapeDtypeStruct((N_ACC, NCLASS), jnp.float32),
    )(p3, b3p)


def kernel(x, adj_t, W1, b1, W2, b2, W3, b3):
    x_ch = x.reshape(N_NODES, 4, W).transpose(1, 0, 2)

    src = adj_t[0]
    dst = adj_t[1]
    npad_e = E_PAD - E_EDGES
    pad_i = jnp.arange(npad_e, dtype=jnp.int32)
    pad_src = (pad_i * 97) % N_NODES
    pad_dst = N_NODES + pad_i % (N_ACC - N_NODES)
    src_b = jnp.concatenate([src, pad_src]).reshape(NB, EB)
    dst_b = jnp.concatenate([dst, pad_dst]).reshape(NB, EB)
    edges_b = jnp.stack([src_b, dst_b], axis=1)

    W3p = jnp.pad(W3, ((0, 0), (0, CW - NCLASS)))
    b3p = jnp.pad(b3, ((0, CW - NCLASS),)).reshape(1, CW)
    zeros = jnp.zeros((EB, W), jnp.float32)
    zeros_cw = jnp.zeros((EB, CW), jnp.float32)

    p1 = _prop_l1(x_ch, edges_b, zeros)
    h1 = _tc_matmul1(p1, W1, b1)
    p2 = _prop_l2(h1, edges_b, zeros)
    t3 = _tc_matmul2(p2, W2, b2, W3p)
    p3 = _prop_l3(t3.reshape(1, N_ACC, CW), edges_b, zeros_cw)
    out = _tc_logsoftmax(p3, b3p)
    return out[:N_NODES]


# speedup vs baseline: 1.1933x; 1.1933x over previous
import functools

import jax
import jax.numpy as jnp
from jax import lax
from jax.experimental import pallas as pl
from jax.experimental.pallas import tpu as pltpu
from jax.experimental.pallas import tpu_sc as plsc

N_NODES = 10000
N_ACC = 10112
E_EDGES = 160000
EB = 128
NB = 1280
E_PAD = NB * EB
NFEAT = 256
NHID = 512
NCLASS = 47
W = 128
CW = 64
ROWS_PER_TILE = N_ACC // 16
BM = 632


def _make_propagate(n_chunks, split_edges, n_out, wp=W, tc_tiling=True):
    mesh = plsc.VectorSubcoreMesh(core_axis_name="c", subcore_axis_name="s")
    count = NB // 32 if split_edges else NB // 16
    params = pltpu.CompilerParams(use_tc_tiling_on_sc=tc_tiling)

    @functools.partial(
        pl.kernel,
        mesh=mesh,
        compiler_params=params,
        out_type=jax.ShapeDtypeStruct((n_out, N_ACC, wp), jnp.float32),
        scratch_types=[
            pltpu.VMEM((2, EB), jnp.int32),
            pltpu.VMEM((2, EB), jnp.int32),
            pltpu.VMEM((EB, wp), jnp.float32),
            pltpu.VMEM((EB, wp), jnp.float32),
            pltpu.VMEM_SHARED((N_ACC, wp), jnp.float32),
            pltpu.SemaphoreType.DMA,
            pltpu.SemaphoreType.DMA,
            pltpu.SemaphoreType.DMA,
        ],
    )
    def prop(table, edges_b, zeros_hbm, out,
             idxA, idxB, rowsA, rowsB, accum,
             semA, semB, semI):
        cid = lax.axis_index("c")
        sid = lax.axis_index("s")
        if split_edges:
            base = cid * (NB // 2) + sid * count
        else:
            base = sid * count
        row0 = sid * ROWS_PER_TILE
        hops = [(0, 128), (128, 128), (256, 128), (384, 128), (512, 120)]

        def wait_rows(buf, sem):
            pltpu.make_async_copy(zeros_hbm, buf, sem).wait()

        def wait_idx(buf, sem):
            pltpu.make_async_copy(edges_b.at[0], buf, sem).wait()

        for c in range(n_chunks if not split_edges else 2):
            owner = (c % 2) if not split_edges else c

            @pl.when(cid == owner)
            def _round(c=c):
                pltpu.sync_copy(zeros_hbm, rowsA)
                for off, sz in hops:
                    pltpu.sync_copy(rowsA.at[pl.ds(0, sz)],
                                    accum.at[pl.ds(row0 + off, sz)])
                plsc.subcore_barrier()
                tbl = table.at[0 if split_edges else c]
                pltpu.sync_copy(edges_b.at[base], idxA)
                pltpu.sync_copy(edges_b.at[base + 1], idxB)
                pltpu.async_copy(tbl.at[idxA.at[0]], rowsA, semA)

                def pair(i, carry):
                    j2 = jnp.minimum(2 * i + 2, count - 1)
                    j3 = jnp.minimum(2 * i + 3, count - 1)
                    pltpu.async_copy(tbl.at[idxB.at[0]], rowsB, semB)
                    wait_rows(rowsA, semA)
                    pltpu.sync_copy(rowsA, accum.at[idxA.at[1]],
                                    add=True)
                    pltpu.async_copy(edges_b.at[base + j2], idxA, semI)
                    wait_rows(rowsB, semB)
                    wait_idx(idxA, semI)
                    pltpu.async_copy(tbl.at[idxA.at[0]], rowsA, semA)
                    pltpu.sync_copy(rowsB, accum.at[idxB.at[1]],
                                    add=True)
                    pltpu.sync_copy(edges_b.at[base + j3], idxB)
                    return carry

                lax.fori_loop(0, count // 2, pair, 0)
                wait_rows(rowsA, semA)
                plsc.subcore_barrier()
                for off, sz in hops:
                    rows = pl.ds(row0 + off, sz)
                    pltpu.sync_copy(accum.at[rows],
                                    rowsB.at[pl.ds(0, sz)])
                    pltpu.sync_copy(rowsB.at[pl.ds(0, sz)],
                                    out.at[c].at[rows])

    return prop


_prop_l1 = _make_propagate(n_chunks=2, split_edges=False, n_out=2)
_prop_l2 = _make_propagate(n_chunks=4, split_edges=False, n_out=4)
_prop_l3 = _make_propagate(n_chunks=1, split_edges=True, n_out=2,
                           wp=CW, tc_tiling=False)


def _t1_body(p_ref, w_ref, b_ref, o_ref):
    for c in range(4):
        cols = slice(c * W, (c + 1) * W)
        acc = jnp.dot(p_ref[0], w_ref[:W, cols],
                      preferred_element_type=jnp.float32)
        acc = acc + jnp.dot(p_ref[1], w_ref[W:, cols],
                            preferred_element_type=jnp.float32)
        o_ref[c] = jnp.maximum(acc + b_ref[:, cols][0], 0.0)


def _t2_body(p_ref, w2_ref, b2_ref, w3_ref, o_ref):
    acc = jnp.dot(p_ref[0], w2_ref[:W, :], preferred_element_type=jnp.float32)
    for k in range(1, 4):
        acc = acc + jnp.dot(p_ref[k], w2_ref[k * W:(k + 1) * W, :],
                            preferred_element_type=jnp.float32)
    h = jnp.maximum(acc + b2_ref[...], 0.0)
    o_ref[...] = jnp.dot(h, w3_ref[...], preferred_element_type=jnp.float32)


def _t3_body(p_ref, b_ref, o_ref):
    s = p_ref[0] + p_ref[1] + b_ref[...]
    col = lax.broadcasted_iota(jnp.int32, s.shape, 1)
    valid = col < NCLASS
    m = jnp.max(jnp.where(valid, s, -1e30), axis=1, keepdims=True)
    e = jnp.where(valid, jnp.exp(s - m), 0.0)
    lse = jnp.log(jnp.sum(e, axis=1, keepdims=True)) + m
    o_ref[...] = (s - lse)[:, :NCLASS]


def _tc_matmul1(p1, W1, b1):
    grid = (N_ACC // BM,)
    return pl.pallas_call(
        _t1_body,
        grid=grid,
        in_specs=[
            pl.BlockSpec((2, BM, W), lambda m: (0, m, 0)),
            pl.BlockSpec((NFEAT, NHID), lambda m: (0, 0)),
            pl.BlockSpec((1, NHID), lambda m: (0, 0)),
        ],
        out_specs=pl.BlockSpec((4, BM, W), lambda m: (0, m, 0)),
        out_shape=jax.ShapeDtypeStruct((4, N_ACC, W), jnp.float32),
    )(p1, W1, b1.reshape(1, NHID))


def _tc_matmul2(p2, W2, b2, W3p):
    grid = (N_ACC // BM,)
    return pl.pallas_call(
        _t2_body,
        grid=grid,
        in_specs=[
            pl.BlockSpec((4, BM, W), lambda m: (0, m, 0)),
            pl.BlockSpec((NHID, NHID), lambda m: (0, 0)),
            pl.BlockSpec((1, NHID), lambda m: (0, 0)),
            pl.BlockSpec((NHID, CW), lambda m: (0, 0)),
        ],
        out_specs=pl.BlockSpec((BM, CW), lambda m: (m, 0)),
        out_shape=jax.ShapeDtypeStruct((N_ACC, CW), jnp.float32),
    )(p2, W2, b2.reshape(1, NHID), W3p)


def _tc_logsoftmax(p3, b3p):
    grid = (N_ACC // BM,)
    return pl.pallas_call(
        _t3_body,
        grid=grid,
        in_specs=[
            pl.BlockSpec((2, BM, CW), lambda m: (0, m, 0)),
            pl.BlockSpec((1, CW), lambda m: (0, 0)),
        ],
        out_specs=pl.BlockSpec((BM, NCLASS), lambda m: (m, 0)),
        out_shape=jax.ShapeDtypeStruct((N_ACC, NCLASS), jnp.float32),
    )(p3, b3p)


def kernel(x, adj_t, W1, b1, W2, b2, W3, b3):
    x_ch = x.reshape(N_NODES, 2, W).transpose(1, 0, 2)

    src = adj_t[0]
    dst = adj_t[1]
    npad_e = E_PAD - E_EDGES
    pad_i = jnp.arange(npad_e, dtype=jnp.int32)
    pad_src = (pad_i * 97) % N_NODES
    pad_dst = N_NODES + pad_i % (N_ACC - N_NODES)
    src_b = jnp.concatenate([src, pad_src]).reshape(NB, EB)
    dst_b = jnp.concatenate([dst, pad_dst]).reshape(NB, EB)
    edges_b = jnp.stack([src_b, dst_b], axis=1)

    W3p = jnp.pad(W3, ((0, 0), (0, CW - NCLASS)))
    b3p = jnp.pad(b3, ((0, CW - NCLASS),)).reshape(1, CW)
    zeros = jnp.zeros((EB, W), jnp.float32)
    zeros_cw = jnp.zeros((EB, CW), jnp.float32)

    p1 = _prop_l1(x_ch, edges_b, zeros)
    h1 = _tc_matmul1(p1, W1, b1)
    p2 = _prop_l2(h1, edges_b, zeros)
    t3 = _tc_matmul2(p2, W2, b2, W3p)
    p3 = _prop_l3(t3.reshape(1, N_ACC, CW), edges_b, zeros_cw)
    out = _tc_logsoftmax(p3, b3p)
    return out[:N_NODES]
